# Initial kernel scaffold; baseline (speedup 1.0000x reference)
import jax
import jax.numpy as jnp
from jax.experimental import pallas as pl


def kernel(inputs, token_table, pos_table):
    raise NotImplementedError("write your pallas kernel here")


# speedup vs baseline: 3.4662x; 3.4662x over previous
import functools

import jax
import jax.numpy as jnp
from jax import lax
from jax.experimental import pallas as pl
from jax.experimental.pallas import tpu as pltpu
from jax.experimental.pallas import tpu_sc as plsc

SEQ = 200
DIM = 64
NC = 2
NS = 16
NW = NC * NS

R = 2
C = R * SEQ
NG = DIM // 16


def _emb_body(idx_hbm, tok_hbm, pos_hbm, out_hbm, idx_v, rows_v, pos_v, gsem):
    total = idx_hbm.shape[0]
    per_w = total // NW
    nchunks = per_w // C

    wid = lax.axis_index("s") * NC + lax.axis_index("c")
    base_w = wid * per_w

    pltpu.sync_copy(pos_hbm, pos_v)

    def chunk_body(g, carry):
        base = base_w + g * C
        pltpu.sync_copy(idx_hbm.at[pl.ds(base, C)], idx_v)
        pltpu.async_copy(tok_hbm.at[idx_v], rows_v, gsem).wait()

        def add_body(s, c2):
            for d in range(NG):
                p = pos_v[s, pl.ds(d * 16, 16)]
                for r in range(R):
                    j = r * SEQ + s
                    rows_v[j, pl.ds(d * 16, 16)] = (
                        rows_v[j, pl.ds(d * 16, 16)] + p
                    )
            return c2

        lax.fori_loop(0, SEQ, add_body, 0)
        pltpu.sync_copy(rows_v, out_hbm.at[pl.ds(base, C), :])
        return carry

    lax.fori_loop(0, nchunks, chunk_body, 0)


@functools.partial(jax.jit, static_argnames=())
def _emb(idx_flat, token_table, pos_table):
    total = idx_flat.shape[0]
    run = pl.kernel(
        _emb_body,
        out_type=jax.ShapeDtypeStruct((total, DIM), jnp.float32),
        mesh=plsc.VectorSubcoreMesh(core_axis_name="c", subcore_axis_name="s"),
        scratch_types=[
            pltpu.VMEM((C,), jnp.int32),
            pltpu.VMEM((C, DIM), jnp.float32),
            pltpu.VMEM((SEQ, DIM), jnp.float32),
            pltpu.SemaphoreType.DMA,
        ],
        compiler_params=pltpu.CompilerParams(use_tc_tiling_on_sc=False),
    )
    return run(idx_flat, token_table, pos_table)


def kernel(inputs, token_table, pos_table):
    batch, seq = inputs.shape
    flat = inputs.reshape(-1).astype(jnp.int32)
    out = _emb(flat, token_table, pos_table)
    return out.reshape(batch, seq, DIM)


# speedup vs baseline: 4.1993x; 1.2115x over previous
import functools

import jax
import jax.numpy as jnp
from jax import lax
from jax.experimental import pallas as pl
from jax.experimental.pallas import tpu as pltpu
from jax.experimental.pallas import tpu_sc as plsc

SEQ = 200
DIM = 64
NC = 2
NS = 16
NW = NC * NS

R = 2
C = R * SEQ
NG = DIM // 16
NB = 4
K = 2


def _emb_body(idx_hbm, tok_hbm, pos_hbm, out_hbm, idx_v, rows_v, pos_v,
              gsem, osem):
    total = idx_hbm.shape[0]
    per_w = total // NW
    nchunks = per_w // C

    wid = lax.axis_index("s") * NC + lax.axis_index("c")
    base_w = wid * per_w

    pltpu.sync_copy(pos_hbm, pos_v)

    def start_gather(g, b):
        base = base_w + g * C
        pltpu.sync_copy(idx_hbm.at[pl.ds(base, C)], idx_v.at[b])
        pltpu.async_copy(tok_hbm.at[idx_v.at[b]], rows_v.at[b], gsem.at[b])

    def wait_gather(b):
        pltpu.make_async_copy(
            tok_hbm.at[idx_v.at[b]], rows_v.at[b], gsem.at[b]
        ).wait()

    def start_out(g, b):
        base = base_w + g * C
        pltpu.async_copy(
            rows_v.at[b], out_hbm.at[pl.ds(base, C), :], osem.at[b]
        )

    def wait_out(b):
        pltpu.make_async_copy(
            rows_v.at[b], out_hbm.at[pl.ds(0, C), :], osem.at[b]
        ).wait()

    for b in range(K):
        start_gather(b, b)

    def outer(h, carry):
        for b in range(NB):
            g = h * NB + b
            wait_gather(b)

            def add_body(s, c2):
                for d in range(NG):
                    p = pos_v[s, pl.ds(d * 16, 16)]
                    for r in range(R):
                        j = r * SEQ + s
                        rows_v[b, j, pl.ds(d * 16, 16)] = (
                            rows_v[b, j, pl.ds(d * 16, 16)] + p
                        )
                return c2

            lax.fori_loop(0, SEQ, add_body, 0)
            start_out(g, b)

            b2 = (b + K) % NB

            @pl.when(g + K >= NB)
            def _():
                wait_out(b2)

            @pl.when(g + K < nchunks)
            def _():
                start_gather(g + K, b2)
        return carry

    lax.fori_loop(0, nchunks // NB, outer, 0)

    for g in range(nchunks - K, nchunks):
        wait_out(g % NB)


@functools.partial(jax.jit, static_argnames=())
def _emb(idx_flat, token_table, pos_table):
    total = idx_flat.shape[0]
    run = pl.kernel(
        _emb_body,
        out_type=jax.ShapeDtypeStruct((total, DIM), jnp.float32),
        mesh=plsc.VectorSubcoreMesh(core_axis_name="c", subcore_axis_name="s"),
        scratch_types=[
            pltpu.VMEM((NB, C), jnp.int32),
            pltpu.VMEM((NB, C, DIM), jnp.float32),
            pltpu.VMEM((SEQ, DIM), jnp.float32),
            pltpu.SemaphoreType.DMA((NB,)),
            pltpu.SemaphoreType.DMA((NB,)),
        ],
        compiler_params=pltpu.CompilerParams(use_tc_tiling_on_sc=False),
    )
    return run(idx_flat, token_table, pos_table)


def kernel(inputs, token_table, pos_table):
    batch, seq = inputs.shape
    flat = inputs.reshape(-1).astype(jnp.int32)
    out = _emb(flat, token_table, pos_table)
    return out.reshape(batch, seq, DIM)
